# baseline (speedup 1.0000x reference)
import functools

import jax
import jax.numpy as jnp
from jax import lax
from jax.experimental import pallas as pl
from jax.experimental.pallas import tpu as pltpu
from jax.experimental.pallas import tpu_sc as plsc

N = 4096
NK = 2
NC, NS, L = 2, 16, 16
NW = NC * NS
ROWS_W = N // NW
CROWS = 8
NCH = ROWS_W // CROWS
NPAIR = NCH // 2
NVEC = CROWS * N // L

NB1, SH1 = 512, 21
NB2, SH2, M2 = 2048, 10, 2047
NB3, M3 = 1024, 1023

_UNROLL = 16


def _mesh():
    return plsc.VectorSubcoreMesh(
        core_axis_name="c", subcore_axis_name="s",
        num_cores=NC, num_subcores=NS)


def _wid():
    return lax.axis_index("s") * NC + lax.axis_index("c")


def _zero(ref, nwords):
    z = jnp.zeros((L,), jnp.int32)

    @plsc.parallel_loop(0, nwords // L, unroll=4)
    def _(i):
        ref[pl.ds(i * L, L)] = z


def _make_hist_pass(nbins, idxshift, idxmask, preshift):
    @functools.partial(
        pl.kernel,
        out_type=[jax.ShapeDtypeStruct((NW, nbins * L), jnp.int32)],
        mesh=_mesh(),
        compiler_params=pltpu.CompilerParams(needs_layout_passes=False),
        scratch_types=[
            pltpu.VMEM((2, CROWS, N), jnp.float32),
            pltpu.VMEM((nbins * L,), jnp.int32),
            pltpu.VMEM((L,), jnp.int32),
            pltpu.VMEM((L,), jnp.int32),
            pltpu.SemaphoreType.DMA,
            pltpu.SemaphoreType.DMA,
        ],
    )
    def _pass(adj_hbm, k_hbm, p_hbm, hist_out, abuf, hist, kbuf, pbuf, s0, s1):
        wid = _wid()
        row_w = wid * ROWS_W
        _zero(hist, nbins * L)
        pltpu.sync_copy(k_hbm, kbuf)
        kk = jnp.max(kbuf[...])
        pltpu.sync_copy(p_hbm, pbuf)
        pv = pbuf[...]
        lane = lax.iota(jnp.int32, L)
        ones = jnp.ones((L,), jnp.int32)
        sems = (s0, s1)

        def start(c, b):
            row = pl.multiple_of(row_w + c * CROWS, 8)
            pltpu.async_copy(
                adj_hbm.at[kk, pl.ds(row, CROWS)], abuf.at[b], sems[b])

        def wait(b):
            pltpu.make_async_copy(
                adj_hbm.at[0, pl.ds(0, CROWS)], abuf.at[b], sems[b]).wait()

        def compute(b):
            @plsc.parallel_loop(0, NVEC, unroll=_UNROLL)
            def _(i):
                r = i >> 8
                col = (i & 255) * L
                a = abuf[b, r, pl.ds(col, L)]
                u = lax.bitcast_convert_type(a, jnp.int32)
                if preshift is None:
                    slot = (u >> idxshift) * L + lane
                    plsc.addupdate_scatter(hist, [slot], ones)
                else:
                    slot = ((u >> idxshift) & idxmask) * L + lane
                    m = (u >> preshift) == pv
                    plsc.addupdate_scatter(hist, [slot], ones, mask=m)

        start(0, 0)

        def pair(j, carry):
            start(2 * j + 1, 1)
            wait(0)
            compute(0)
            start(2 * j + 2, 0)
            wait(1)
            compute(1)
            return carry

        lax.fori_loop(0, NPAIR - 1, pair, 0)
        start(NCH - 1, 1)
        wait(0)
        compute(0)
        wait(1)
        compute(1)

        pltpu.sync_copy(hist, hist_out.at[wid])

    return _pass


_pass1 = _make_hist_pass(NB1, SH1, 0x7FF, None)
_pass2 = _make_hist_pass(NB2, SH2, M2, SH1)
_pass3 = _make_hist_pass(NB3, 0, M3, SH2)


_CROWS_TC = 256


def _count_body(e_ref, o_ref):
    @pl.when(pl.program_id(0) == 0)
    def _():
        o_ref[...] = jnp.zeros((1, 1), jnp.int32)

    part = jnp.sum((jnp.abs(e_ref[...]) > 0.0).astype(jnp.int32))
    o_ref[...] += part[None, None]


def _count(edge):
    return pl.pallas_call(
        _count_body,
        grid=(N // _CROWS_TC,),
        in_specs=[pl.BlockSpec((_CROWS_TC, N), lambda i: (i, 0))],
        out_specs=pl.BlockSpec((1, 1), lambda i: (0, 0)),
        out_shape=jax.ShapeDtypeStruct((1, 1), jnp.int32),
    )(edge)


_ROWS = 256


def _final_body(s_ref, e_ref, a_ref, o_ref):
    thre = lax.bitcast_convert_type(s_ref[1], jnp.float32)
    mask = (jnp.abs(a_ref[0]) < thre).astype(jnp.float32)
    o_ref[...] = e_ref[...] + mask


def _final(kpat, edge, adj):
    grid_spec = pltpu.PrefetchScalarGridSpec(
        num_scalar_prefetch=1,
        grid=(N // _ROWS,),
        in_specs=[
            pl.BlockSpec((_ROWS, N), lambda i, s: (i, 0)),
            pl.BlockSpec((1, _ROWS, N), lambda i, s: (s[0], i, 0)),
        ],
        out_specs=pl.BlockSpec((_ROWS, N), lambda i, s: (i, 0)),
    )
    return pl.pallas_call(
        _final_body,
        grid_spec=grid_spec,
        out_shape=jax.ShapeDtypeStruct((N, N), jnp.float32),
    )(kpat, edge, adj)


def _pick(hist_lane_private, q):
    nbins = hist_lane_private.shape[1] // L
    hist = hist_lane_private.reshape(NW, nbins, L).sum(axis=(0, 2))
    cum = jnp.cumsum(hist)
    below = cum <= q
    b = jnp.minimum(jnp.sum(below.astype(jnp.int32)), nbins - 1)
    q_next = q - jnp.sum(jnp.where(below, hist, 0))
    return b.astype(jnp.int32), q_next


def kernel(edge_index, n, num_sample, k, adj_mask1_train, rate):
    kvec = jnp.full((L,), k, jnp.int32)
    zvec = jnp.zeros((L,), jnp.int32)

    (h1,) = _pass1(adj_mask1_train, kvec, zvec)
    nonzero = _count(edge_index)[0, 0]
    q = (nonzero.astype(jnp.float32) * rate).astype(jnp.int32)

    b1, q1 = _pick(h1, q)
    (h2,) = _pass2(adj_mask1_train, kvec, jnp.broadcast_to(b1, (L,)))
    b2, q2 = _pick(h2, q1)
    p2 = b1 * NB2 + b2
    (h3,) = _pass3(adj_mask1_train, kvec, jnp.broadcast_to(p2, (L,)))
    b3, _ = _pick(h3, q2)

    pat = (b1 << SH1) | (b2 << SH2) | b3
    kpat = jnp.stack([k.astype(jnp.int32), pat])
    return _final(kpat, edge_index, adj_mask1_train)


# speedup vs baseline: 1.2381x; 1.2381x over previous
import functools

import jax
import jax.numpy as jnp
from jax import lax
from jax.experimental import pallas as pl
from jax.experimental.pallas import tpu as pltpu
from jax.experimental.pallas import tpu_sc as plsc

N = 4096
NK = 2
NC, NS, L = 2, 16, 16
NW = NC * NS
ROWS_W = N // NW

NBA = 32768
NBB = 32768
SH = 15
MB = NBB - 1

_UNROLL = 16


def _mesh():
    return plsc.VectorSubcoreMesh(
        core_axis_name="c", subcore_axis_name="s",
        num_cores=NC, num_subcores=NS)


def _wid():
    return lax.axis_index("s") * NC + lax.axis_index("c")


def _zero(ref, nwords):
    z = jnp.zeros((L,), jnp.int32)

    @plsc.parallel_loop(0, nwords // L, unroll=4)
    def _(i):
        ref[pl.ds(i * L, L)] = z


def _make_hist_pass(nbins, masked, crows):
    nch = ROWS_W // crows
    npair = nch // 2
    nvec = crows * N // L

    @functools.partial(
        pl.kernel,
        out_type=[jax.ShapeDtypeStruct((NW, nbins), jnp.int32)],
        mesh=_mesh(),
        compiler_params=pltpu.CompilerParams(needs_layout_passes=False),
        scratch_types=[
            pltpu.VMEM((2, crows, N), jnp.float32),
            pltpu.VMEM((nbins,), jnp.int32),
            pltpu.VMEM((L,), jnp.int32),
            pltpu.VMEM((L,), jnp.int32),
            pltpu.SemaphoreType.DMA,
            pltpu.SemaphoreType.DMA,
        ],
    )
    def _pass(adj_hbm, k_hbm, p_hbm, hist_out, abuf, hist, kbuf, pbuf, s0, s1):
        wid = _wid()
        row_w = wid * ROWS_W
        _zero(hist, nbins)
        pltpu.sync_copy(k_hbm, kbuf)
        kk = jnp.max(kbuf[...])
        pltpu.sync_copy(p_hbm, pbuf)
        pv = pbuf[...]
        ones = jnp.ones((L,), jnp.int32)
        sems = (s0, s1)

        def start(c, b):
            row = pl.multiple_of(row_w + c * crows, 8)
            pltpu.async_copy(
                adj_hbm.at[kk, pl.ds(row, crows)], abuf.at[b], sems[b])

        def wait(b):
            pltpu.make_async_copy(
                adj_hbm.at[0, pl.ds(0, crows)], abuf.at[b], sems[b]).wait()

        def compute(b):
            @plsc.parallel_loop(0, nvec, unroll=_UNROLL)
            def _(i):
                r = i >> 8
                col = (i & 255) * L
                a = abuf[b, r, pl.ds(col, L)]
                u = lax.bitcast_convert_type(a, jnp.int32)
                if not masked:
                    plsc.addupdate_scatter(hist, [u >> SH], ones)
                else:
                    m = (u >> SH) == pv
                    plsc.addupdate_scatter(
                        hist, [u & MB], ones, mask=m)

        start(0, 0)

        def pair(j, carry):
            start(2 * j + 1, 1)
            wait(0)
            compute(0)
            start(2 * j + 2, 0)
            wait(1)
            compute(1)
            return carry

        lax.fori_loop(0, npair - 1, pair, 0)
        start(nch - 1, 1)
        wait(0)
        compute(0)
        wait(1)
        compute(1)

        pltpu.sync_copy(hist, hist_out.at[wid])

    return _pass


_passA = _make_hist_pass(NBA, False, 8)
_passB = _make_hist_pass(NBB, True, 8)


_CROWS_TC = 256


def _count_body(e_ref, o_ref):
    @pl.when(pl.program_id(0) == 0)
    def _():
        o_ref[...] = jnp.zeros((1, 1), jnp.int32)

    part = jnp.sum((jnp.abs(e_ref[...]) > 0.0).astype(jnp.int32))
    o_ref[...] += part[None, None]


def _count(edge):
    return pl.pallas_call(
        _count_body,
        grid=(N // _CROWS_TC,),
        in_specs=[pl.BlockSpec((_CROWS_TC, N), lambda i: (i, 0))],
        out_specs=pl.BlockSpec((1, 1), lambda i: (0, 0)),
        out_shape=jax.ShapeDtypeStruct((1, 1), jnp.int32),
    )(edge)


_ROWS = 256


def _final_body(s_ref, e_ref, a_ref, o_ref):
    thre = lax.bitcast_convert_type(s_ref[1], jnp.float32)
    mask = (jnp.abs(a_ref[0]) < thre).astype(jnp.float32)
    o_ref[...] = e_ref[...] + mask


def _final(kpat, edge, adj):
    grid_spec = pltpu.PrefetchScalarGridSpec(
        num_scalar_prefetch=1,
        grid=(N // _ROWS,),
        in_specs=[
            pl.BlockSpec((_ROWS, N), lambda i, s: (i, 0)),
            pl.BlockSpec((1, _ROWS, N), lambda i, s: (s[0], i, 0)),
        ],
        out_specs=pl.BlockSpec((_ROWS, N), lambda i, s: (i, 0)),
    )
    return pl.pallas_call(
        _final_body,
        grid_spec=grid_spec,
        out_shape=jax.ShapeDtypeStruct((N, N), jnp.float32),
    )(kpat, edge, adj)


def _pick(h, q):
    nbins = h.shape[1]
    hist = h.sum(axis=0)
    cum = jnp.cumsum(hist)
    below = cum <= q
    b = jnp.minimum(jnp.sum(below.astype(jnp.int32)), nbins - 1)
    q_next = q - jnp.sum(jnp.where(below, hist, 0))
    return b.astype(jnp.int32), q_next


def kernel(edge_index, n, num_sample, k, adj_mask1_train, rate):
    kvec = jnp.full((L,), k, jnp.int32)
    zvec = jnp.zeros((L,), jnp.int32)

    (ha,) = _passA(adj_mask1_train, kvec, zvec)
    nonzero = _count(edge_index)[0, 0]
    q = (nonzero.astype(jnp.float32) * rate).astype(jnp.int32)

    ba, qa = _pick(ha, q)
    (hb,) = _passB(adj_mask1_train, kvec, jnp.broadcast_to(ba, (L,)))
    bb, _ = _pick(hb, qa)

    pat = (ba << SH) | bb
    kpat = jnp.stack([k.astype(jnp.int32), pat])
    return _final(kpat, edge_index, adj_mask1_train)
